# Initial kernel scaffold; baseline (speedup 1.0000x reference)
import jax
import jax.numpy as jnp
from jax.experimental import pallas as pl


def kernel(feat, img_size, W1, b1, W_cls, b_cls, W_loc, b_loc):
    raise NotImplementedError("write your pallas kernel here")


# speedup vs baseline: 36.8609x; 36.8609x over previous
import jax
import jax.numpy as jnp
import numpy as np
from jax.experimental import pallas as pl
from jax.experimental.pallas import tpu as pltpu

_FEAT_STRIDE = 16
_SCALES = (8.0, 16.0, 32.0)
_RATIOS = (0.5, 1.0, 2.0)
_N_PRE = 2000
_N_POST = 300
_NMS_THRESH = 0.7
_MIN_SIZE = 16.0

_R = 72
_C = 128


def _anchors_np(h, w):
    base = float(_FEAT_STRIDE)
    ctr = base / 2.0
    aa = []
    for r in _RATIOS:
        for s in _SCALES:
            ah = base * s * np.sqrt(r)
            aw = base * s * np.sqrt(1.0 / r)
            aa.append([ctr - ah / 2.0, ctr - aw / 2.0, ctr + ah / 2.0, ctr + aw / 2.0])
    base_anchors = np.asarray(aa, dtype=np.float32)
    sy = np.arange(h, dtype=np.float32) * base
    sx = np.arange(w, dtype=np.float32) * base
    gy, gx = np.meshgrid(sy, sx, indexing='ij')
    shifts = np.stack([gy.ravel(), gx.ravel(), gy.ravel(), gx.ravel()], axis=1)
    return (shifts[:, None, :] + base_anchors[None, :, :]).reshape(-1, 4)


def _conv_body(xp_ref, w1_ref, b1_ref, wc_ref, bc_ref, wl_ref, bl_ref,
               loc_ref, cls_ref):
    xw = xp_ref[0]
    acc = jnp.zeros((1024, 512), dtype=jnp.float32)
    for ky in range(3):
        for kx in range(3):
            xs = xw[ky:ky + 32, kx:kx + 32, :].reshape(1024, 512)
            wk = w1_ref[(ky * 3 + kx) * 512:(ky * 3 + kx + 1) * 512, :]
            acc = acc + jnp.dot(xs, wk, preferred_element_type=jnp.float32)
    h = jnp.maximum(acc + b1_ref[0], 0.0)
    loc_ref[0] = jnp.dot(h, wl_ref[...], preferred_element_type=jnp.float32) + bl_ref[0]
    cls_ref[0] = jnp.dot(h, wc_ref[...], preferred_element_type=jnp.float32) + bc_ref[0]


def _count_gt(keys, t):
    return jnp.sum((keys > t).astype(jnp.int32))


def _proposal_body(dy_ref, dx_ref, dh_ref, dw_ref, s0_ref, s1_ref,
                   ay1_ref, ax1_ref, ay2_ref, ax2_ref, img_ref, out_ref):
    ih = img_ref[0]
    iw = img_ref[1]
    dy, dx, dh, dw = dy_ref[0], dx_ref[0], dh_ref[0], dw_ref[0]
    ay1, ax1, ay2, ax2 = ay1_ref[...], ax1_ref[...], ay2_ref[...], ax2_ref[...]

    src_h = ay2 - ay1
    src_w = ax2 - ax1
    src_cy = ay1 + 0.5 * src_h
    src_cx = ax1 + 0.5 * src_w
    cy = dy * src_h + src_cy
    cx = dx * src_w + src_cx
    hh = jnp.exp(dh) * src_h
    ww = jnp.exp(dw) * src_w
    y1 = jnp.clip(cy - 0.5 * hh, 0.0, ih)
    x1 = jnp.clip(cx - 0.5 * ww, 0.0, iw)
    y2 = jnp.clip(cy + 0.5 * hh, 0.0, ih)
    x2 = jnp.clip(cx + 0.5 * ww, 0.0, iw)

    s0, s1 = s0_ref[0], s1_ref[0]
    m = jnp.maximum(s0, s1)
    e0 = jnp.exp(s0 - m)
    e1 = jnp.exp(s1 - m)
    fg = e1 / (e0 + e1)
    hs = y2 - y1
    ws = x2 - x1
    valid = (hs >= _MIN_SIZE) & (ws >= _MIN_SIZE)
    fg = jnp.where(valid, fg, -jnp.inf)

    keys = jax.lax.bitcast_convert_type(fg, jnp.int32)
    k = jnp.int32(_N_PRE)

    def bis_body(_, lohi):
        lo, hi = lohi
        mid = (lo + hi) // 2
        c = _count_gt(keys, mid)
        return jnp.where(c >= k, mid, lo), jnp.where(c >= k, hi, mid)

    lo0 = jnp.int32(-8388609)
    hi0 = jnp.int32(1 << 30)
    lo, hi = jax.lax.fori_loop(0, 31, bis_body, (lo0, hi0))
    tstar = hi
    c_gt = _count_gt(keys, tstar)
    r = k - c_gt

    tie = keys == tstar
    n_tie = jnp.sum(tie.astype(jnp.int32))
    flat = (jax.lax.broadcasted_iota(jnp.int32, (_R, _C), 0) * _C
            + jax.lax.broadcasted_iota(jnp.int32, (_R, _C), 1))

    def tie_body(_, lohi):
        lo, hi = lohi
        mid = (lo + hi) // 2
        g = jnp.sum((tie & (flat < mid)).astype(jnp.int32))
        return jnp.where(g >= r + 1, lo, mid), jnp.where(g >= r + 1, mid, hi)

    tlo, thi = jax.lax.fori_loop(0, 15, tie_body, (jnp.int32(0), jnp.int32(_R * _C + 2)))
    u0 = thi
    tie_elig = tie & ((n_tie <= r) | (flat < u0 - 1))
    elig = (keys > tstar) | tie_elig

    area = hs * ws
    neg_inf = jnp.float32(-jnp.inf)
    big = jnp.int32(1 << 30)

    def select(active):
        mval = jnp.max(jnp.where(active, fg, neg_inf))
        cand = active & (fg == mval)
        sel = jnp.min(jnp.where(cand, flat, big))
        return sel

    def extract(arr, sel):
        return jnp.sum(jnp.where(flat == sel, arr, 0.0))

    sel0 = select(elig)
    fb = (extract(y1, sel0), extract(x1, sel0), extract(y2, sel0), extract(x2, sel0))

    def nms_body(t, carry):
        ai = carry
        active = ai != 0
        anyact = jnp.any(active)
        sel = select(active)
        by1 = jnp.where(anyact, extract(y1, sel), fb[0])
        bx1 = jnp.where(anyact, extract(x1, sel), fb[1])
        by2 = jnp.where(anyact, extract(y2, sel), fb[2])
        bx2 = jnp.where(anyact, extract(x2, sel), fb[3])
        out_ref[0, pl.ds(t, 1), :] = jnp.stack([by1, bx1, by2, bx2]).reshape(1, 4)
        sarea = (by2 - by1) * (bx2 - bx1)
        ihh = jnp.clip(jnp.minimum(y2, by2) - jnp.maximum(y1, by1), 0.0, None)
        iww = jnp.clip(jnp.minimum(x2, bx2) - jnp.maximum(x1, bx1), 0.0, None)
        inter = ihh * iww
        iou = inter / (area + sarea - inter + 1e-9)
        suppress = (iou > _NMS_THRESH) | (flat == sel)
        return jnp.where(suppress, 0, ai) * anyact.astype(jnp.int32)

    jax.lax.fori_loop(0, _N_POST, nms_body, elig.astype(jnp.int32))


def kernel(feat, img_size, W1, b1, W_cls, b_cls, W_loc, b_loc):
    B, Cin, H, W = feat.shape
    n_anch = H * W * 9

    x = feat.transpose(0, 2, 3, 1)
    xp = jnp.pad(x, ((0, 0), (1, 1), (1, 1), (0, 0)))
    w1r = W1.transpose(2, 3, 1, 0).reshape(9 * Cin, 512)
    wc = W_cls.transpose(2, 3, 1, 0).reshape(512, 18)
    wl = W_loc.transpose(2, 3, 1, 0).reshape(512, 36)
    b1r = b1.reshape(1, 512)
    bcr = b_cls.reshape(1, 18)
    blr = b_loc.reshape(1, 36)

    loc_out, cls_out = pl.pallas_call(
        _conv_body,
        grid=(B,),
        in_specs=[
            pl.BlockSpec((1, H + 2, W + 2, Cin), lambda i: (i, 0, 0, 0)),
            pl.BlockSpec((9 * Cin, 512), lambda i: (0, 0)),
            pl.BlockSpec((1, 512), lambda i: (0, 0)),
            pl.BlockSpec((512, 18), lambda i: (0, 0)),
            pl.BlockSpec((1, 18), lambda i: (0, 0)),
            pl.BlockSpec((512, 36), lambda i: (0, 0)),
            pl.BlockSpec((1, 36), lambda i: (0, 0)),
        ],
        out_specs=[
            pl.BlockSpec((1, H * W, 36), lambda i: (i, 0, 0)),
            pl.BlockSpec((1, H * W, 18), lambda i: (i, 0, 0)),
        ],
        out_shape=[
            jax.ShapeDtypeStruct((B, H * W, 36), jnp.float32),
            jax.ShapeDtypeStruct((B, H * W, 18), jnp.float32),
        ],
    )(xp, w1r, b1r, wc, bcr, wl, blr)

    rpn_loc = loc_out.reshape(B, n_anch, 4)
    rpn_scores = cls_out.reshape(B, n_anch, 2)

    anch = _anchors_np(H, W)
    ay1 = jnp.asarray(anch[:, 0].reshape(_R, _C))
    ax1 = jnp.asarray(anch[:, 1].reshape(_R, _C))
    ay2 = jnp.asarray(anch[:, 2].reshape(_R, _C))
    ax2 = jnp.asarray(anch[:, 3].reshape(_R, _C))

    dy = rpn_loc[:, :, 0].reshape(B, _R, _C)
    dx = rpn_loc[:, :, 1].reshape(B, _R, _C)
    dh = rpn_loc[:, :, 2].reshape(B, _R, _C)
    dw = rpn_loc[:, :, 3].reshape(B, _R, _C)
    s0 = rpn_scores[:, :, 0].reshape(B, _R, _C)
    s1 = rpn_scores[:, :, 1].reshape(B, _R, _C)
    imgf = img_size.astype(jnp.float32)

    per_img = pl.BlockSpec((1, _R, _C), lambda i: (i, 0, 0))
    shared = pl.BlockSpec((_R, _C), lambda i: (0, 0))

    rois_out = pl.pallas_call(
        _proposal_body,
        grid=(B,),
        in_specs=[per_img, per_img, per_img, per_img, per_img, per_img,
                  shared, shared, shared, shared,
                  pl.BlockSpec(memory_space=pltpu.SMEM)],
        out_specs=pl.BlockSpec((1, _N_POST, 4), lambda i: (i, 0, 0)),
        out_shape=jax.ShapeDtypeStruct((B, _N_POST, 4), jnp.float32),
    )(dy, dx, dh, dw, s0, s1, ay1, ax1, ay2, ax2, imgf)

    rois = rois_out.reshape(B * _N_POST, 4)
    roi_indices = jnp.repeat(jnp.arange(B, dtype=jnp.int32), _N_POST)
    return rpn_loc, rpn_scores, rois, roi_indices, jnp.asarray(anch)


# speedup vs baseline: 37.0541x; 1.0052x over previous
import jax
import jax.numpy as jnp
import numpy as np
from jax.experimental import pallas as pl
from jax.experimental.pallas import tpu as pltpu

_FEAT_STRIDE = 16
_SCALES = (8.0, 16.0, 32.0)
_RATIOS = (0.5, 1.0, 2.0)
_N_PRE = 2000
_N_POST = 300
_NMS_THRESH = 0.7
_MIN_SIZE = 16.0

_R = 72
_C = 128


def _anchors_np(h, w):
    base = float(_FEAT_STRIDE)
    ctr = base / 2.0
    aa = []
    for r in _RATIOS:
        for s in _SCALES:
            ah = base * s * np.sqrt(r)
            aw = base * s * np.sqrt(1.0 / r)
            aa.append([ctr - ah / 2.0, ctr - aw / 2.0, ctr + ah / 2.0, ctr + aw / 2.0])
    base_anchors = np.asarray(aa, dtype=np.float32)
    sy = np.arange(h, dtype=np.float32) * base
    sx = np.arange(w, dtype=np.float32) * base
    gy, gx = np.meshgrid(sy, sx, indexing='ij')
    shifts = np.stack([gy.ravel(), gx.ravel(), gy.ravel(), gx.ravel()], axis=1)
    return (shifts[:, None, :] + base_anchors[None, :, :]).reshape(-1, 4)


def _conv_body(xp_ref, w1_ref, b1_ref, h_ref):
    xw = xp_ref[0]
    xcols = jnp.concatenate(
        [xw[ky:ky + 32, kx:kx + 32, :].reshape(1024, 512)
         for ky in range(3) for kx in range(3)], axis=1)
    acc = jnp.dot(xcols, w1_ref[...], preferred_element_type=jnp.float32)
    h_ref[0] = jnp.maximum(acc + b1_ref[0], 0.0)


def _heads_body(h_ref, wc_ref, bc_ref, wl_ref, bl_ref, loc_ref, cls_ref):
    h = h_ref[0]
    loc_ref[0] = jnp.dot(h, wl_ref[...], preferred_element_type=jnp.float32) + bl_ref[0]
    cls_ref[0] = jnp.dot(h, wc_ref[...], preferred_element_type=jnp.float32) + bc_ref[0]


def _count_gt(keys, t):
    return jnp.sum((keys > t).astype(jnp.int32))


def _proposal_body(dy_ref, dx_ref, dh_ref, dw_ref, s0_ref, s1_ref,
                   ay1_ref, ax1_ref, ay2_ref, ax2_ref, img_ref, out_ref):
    ih = img_ref[0]
    iw = img_ref[1]
    dy, dx, dh, dw = dy_ref[0], dx_ref[0], dh_ref[0], dw_ref[0]
    ay1, ax1, ay2, ax2 = ay1_ref[...], ax1_ref[...], ay2_ref[...], ax2_ref[...]

    src_h = ay2 - ay1
    src_w = ax2 - ax1
    src_cy = ay1 + 0.5 * src_h
    src_cx = ax1 + 0.5 * src_w
    cy = dy * src_h + src_cy
    cx = dx * src_w + src_cx
    hh = jnp.exp(dh) * src_h
    ww = jnp.exp(dw) * src_w
    y1 = jnp.clip(cy - 0.5 * hh, 0.0, ih)
    x1 = jnp.clip(cx - 0.5 * ww, 0.0, iw)
    y2 = jnp.clip(cy + 0.5 * hh, 0.0, ih)
    x2 = jnp.clip(cx + 0.5 * ww, 0.0, iw)

    s0, s1 = s0_ref[0], s1_ref[0]
    m = jnp.maximum(s0, s1)
    e0 = jnp.exp(s0 - m)
    e1 = jnp.exp(s1 - m)
    fg = e1 / (e0 + e1)
    hs = y2 - y1
    ws = x2 - x1
    valid = (hs >= _MIN_SIZE) & (ws >= _MIN_SIZE)
    fg = jnp.where(valid, fg, -jnp.inf)

    keys = jax.lax.bitcast_convert_type(fg, jnp.int32)
    k = jnp.int32(_N_PRE)

    def bis_body(_, lohi):
        lo, hi = lohi
        mid = (lo + hi) // 2
        c = _count_gt(keys, mid)
        return jnp.where(c >= k, mid, lo), jnp.where(c >= k, hi, mid)

    lo0 = jnp.int32(-8388609)
    hi0 = jnp.int32(1 << 30)
    lo, hi = jax.lax.fori_loop(0, 31, bis_body, (lo0, hi0))
    tstar = hi
    c_gt = _count_gt(keys, tstar)
    r = k - c_gt

    tie = keys == tstar
    n_tie = jnp.sum(tie.astype(jnp.int32))
    flat = (jax.lax.broadcasted_iota(jnp.int32, (_R, _C), 0) * _C
            + jax.lax.broadcasted_iota(jnp.int32, (_R, _C), 1))

    def tie_body(_, lohi):
        lo, hi = lohi
        mid = (lo + hi) // 2
        g = jnp.sum((tie & (flat < mid)).astype(jnp.int32))
        return jnp.where(g >= r + 1, lo, mid), jnp.where(g >= r + 1, mid, hi)

    tlo, thi = jax.lax.fori_loop(0, 15, tie_body, (jnp.int32(0), jnp.int32(_R * _C + 2)))
    u0 = thi
    tie_elig = tie & ((n_tie <= r) | (flat < u0 - 1))
    elig = (keys > tstar) | tie_elig

    area = hs * ws
    neg_inf = jnp.float32(-jnp.inf)
    big = jnp.int32(1 << 30)

    def select(active):
        mval = jnp.max(jnp.where(active, fg, neg_inf))
        cand = active & (fg == mval)
        sel = jnp.min(jnp.where(cand, flat, big))
        return sel

    def extract(arr, sel):
        return jnp.sum(jnp.where(flat == sel, arr, 0.0))

    sel0 = select(elig)
    fb = (extract(y1, sel0), extract(x1, sel0), extract(y2, sel0), extract(x2, sel0))

    def nms_body(t, carry):
        ai = carry
        active = ai != 0
        anyact = jnp.any(active)
        sel = select(active)
        by1 = jnp.where(anyact, extract(y1, sel), fb[0])
        bx1 = jnp.where(anyact, extract(x1, sel), fb[1])
        by2 = jnp.where(anyact, extract(y2, sel), fb[2])
        bx2 = jnp.where(anyact, extract(x2, sel), fb[3])
        out_ref[0, pl.ds(t, 1), :] = jnp.stack([by1, bx1, by2, bx2]).reshape(1, 4)
        sarea = (by2 - by1) * (bx2 - bx1)
        ihh = jnp.clip(jnp.minimum(y2, by2) - jnp.maximum(y1, by1), 0.0, None)
        iww = jnp.clip(jnp.minimum(x2, bx2) - jnp.maximum(x1, bx1), 0.0, None)
        inter = ihh * iww
        iou = inter / (area + sarea - inter + 1e-9)
        suppress = (iou > _NMS_THRESH) | (flat == sel)
        return jnp.where(suppress, 0, ai) * anyact.astype(jnp.int32)

    jax.lax.fori_loop(0, _N_POST, nms_body, elig.astype(jnp.int32))


def kernel(feat, img_size, W1, b1, W_cls, b_cls, W_loc, b_loc):
    B, Cin, H, W = feat.shape
    n_anch = H * W * 9

    x = feat.transpose(0, 2, 3, 1)
    xp = jnp.pad(x, ((0, 0), (1, 1), (1, 1), (0, 0)))
    w1r = W1.transpose(2, 3, 1, 0).reshape(9 * Cin, 512)
    wc = W_cls.transpose(2, 3, 1, 0).reshape(512, 18)
    wl = W_loc.transpose(2, 3, 1, 0).reshape(512, 36)
    b1r = b1.reshape(1, 512)
    bcr = b_cls.reshape(1, 18)
    blr = b_loc.reshape(1, 36)

    h_out = pl.pallas_call(
        _conv_body,
        grid=(B,),
        in_specs=[
            pl.BlockSpec((1, H + 2, W + 2, Cin), lambda i: (i, 0, 0, 0)),
            pl.BlockSpec((9 * Cin, 512), lambda i: (0, 0)),
            pl.BlockSpec((1, 512), lambda i: (0, 0)),
        ],
        out_specs=pl.BlockSpec((1, H * W, 512), lambda i: (i, 0, 0)),
        out_shape=jax.ShapeDtypeStruct((B, H * W, 512), jnp.float32),
    )(xp, w1r, b1r)

    loc_out, cls_out = pl.pallas_call(
        _heads_body,
        grid=(B,),
        in_specs=[
            pl.BlockSpec((1, H * W, 512), lambda i: (i, 0, 0)),
            pl.BlockSpec((512, 18), lambda i: (0, 0)),
            pl.BlockSpec((1, 18), lambda i: (0, 0)),
            pl.BlockSpec((512, 36), lambda i: (0, 0)),
            pl.BlockSpec((1, 36), lambda i: (0, 0)),
        ],
        out_specs=[
            pl.BlockSpec((1, H * W, 36), lambda i: (i, 0, 0)),
            pl.BlockSpec((1, H * W, 18), lambda i: (i, 0, 0)),
        ],
        out_shape=[
            jax.ShapeDtypeStruct((B, H * W, 36), jnp.float32),
            jax.ShapeDtypeStruct((B, H * W, 18), jnp.float32),
        ],
    )(h_out, wc, bcr, wl, blr)

    rpn_loc = loc_out.reshape(B, n_anch, 4)
    rpn_scores = cls_out.reshape(B, n_anch, 2)

    anch = _anchors_np(H, W)
    ay1 = jnp.asarray(anch[:, 0].reshape(_R, _C))
    ax1 = jnp.asarray(anch[:, 1].reshape(_R, _C))
    ay2 = jnp.asarray(anch[:, 2].reshape(_R, _C))
    ax2 = jnp.asarray(anch[:, 3].reshape(_R, _C))

    dy = rpn_loc[:, :, 0].reshape(B, _R, _C)
    dx = rpn_loc[:, :, 1].reshape(B, _R, _C)
    dh = rpn_loc[:, :, 2].reshape(B, _R, _C)
    dw = rpn_loc[:, :, 3].reshape(B, _R, _C)
    s0 = rpn_scores[:, :, 0].reshape(B, _R, _C)
    s1 = rpn_scores[:, :, 1].reshape(B, _R, _C)
    imgf = img_size.astype(jnp.float32)

    per_img = pl.BlockSpec((1, _R, _C), lambda i: (i, 0, 0))
    shared = pl.BlockSpec((_R, _C), lambda i: (0, 0))

    rois_out = pl.pallas_call(
        _proposal_body,
        grid=(B,),
        in_specs=[per_img, per_img, per_img, per_img, per_img, per_img,
                  shared, shared, shared, shared,
                  pl.BlockSpec(memory_space=pltpu.SMEM)],
        out_specs=pl.BlockSpec((1, _N_POST, 4), lambda i: (i, 0, 0)),
        out_shape=jax.ShapeDtypeStruct((B, _N_POST, 4), jnp.float32),
    )(dy, dx, dh, dw, s0, s1, ay1, ax1, ay2, ax2, imgf)

    rois = rois_out.reshape(B * _N_POST, 4)
    roi_indices = jnp.repeat(jnp.arange(B, dtype=jnp.int32), _N_POST)
    return rpn_loc, rpn_scores, rois, roi_indices, jnp.asarray(anch)
